# baseline (speedup 1.0000x reference)
import jax
import jax.numpy as jnp
from jax import lax
from jax.experimental import pallas as pl
from jax.experimental.pallas import tpu as pltpu

N, D, K = 18432, 64, 1024
BM = 4608


def _assign_quant_body(z_ref, c_ref, codes_ref, q_ref):
    z = z_ref[...]
    c = c_ref[...]
    d = lax.dot_general(c, z * (-2.0), (((1,), (1,)), ((), ())),
                        preferred_element_type=jnp.float32)
    d = d + jnp.sum(z * z, axis=1)[None, :]
    d = d + jnp.sum(c * c, axis=1)[:, None]
    codes = jnp.argmin(d, axis=0).astype(jnp.int32)
    codes_ref[0, 0, :] = codes
    onehot = (codes[:, None] == lax.broadcasted_iota(jnp.int32, (BM, K), 1))
    q_ref[...] = lax.dot_general(onehot.astype(jnp.float32), c,
                                 (((1,), (0,)), ((), ())),
                                 preferred_element_type=jnp.float32)


@jax.jit
def kernel(z, centers):
    grid = N // BM
    codes3, quant = pl.pallas_call(
        _assign_quant_body,
        grid=(grid,),
        compiler_params=pltpu.CompilerParams(
            vmem_limit_bytes=100 * 1024 * 1024),
        in_specs=[
            pl.BlockSpec((BM, D), lambda i: (i, 0)),
            pl.BlockSpec((K, D), lambda i: (0, 0)),
        ],
        out_specs=[
            pl.BlockSpec((1, 1, BM), lambda i: (i, 0, 0)),
            pl.BlockSpec((BM, D), lambda i: (i, 0)),
        ],
        out_shape=[
            jax.ShapeDtypeStruct((grid, 1, BM), jnp.int32),
            jax.ShapeDtypeStruct((N, D), jnp.float32),
        ],
    )(z, centers)
    return codes3.reshape(N), quant
